# baseline (speedup 1.0000x reference)
import functools

import jax
import jax.numpy as jnp
from jax import lax
from jax.experimental import pallas as pl
from jax.experimental.pallas import tpu as pltpu
from jax.experimental.pallas import tpu_sc as plsc
from jax._src.pallas.mosaic import sc_core

NC = 2
NS = 16
NW = NC * NS
L = 16
STEP = 1.0 / 65536.0

_SC_PARAMS = pltpu.CompilerParams(
    use_tc_tiling_on_sc=False,
    needs_layout_passes=False,
)


def _widx():
    return lax.axis_index("s") * NC + lax.axis_index("c")


def _iota():
    return lax.iota(jnp.int32, L)


_SC_PARAMS_TILED = pltpu.CompilerParams(
    use_tc_tiling_on_sc=True,
    needs_layout_passes=False,
)


def _prep(ut, itt, tail_u, tail_i, nei_users, nei_items):
    D, n = ut.shape
    BU = 256
    full_b = n // BU
    tail = n - full_b * BU
    base_b = full_b // NW
    rem_b = full_b - base_b * NW
    steps = base_b // 2
    Bn = nei_users.shape[0]
    per = Bn // NW
    CH = 128
    nch = per // CH

    mesh = plsc.VectorSubcoreMesh(core_axis_name="c", subcore_axis_name="s")

    @functools.partial(
        pl.kernel,
        out_type=(jax.ShapeDtypeStruct((n * D,), jnp.float32),
                  jax.ShapeDtypeStruct((n * D,), jnp.float32),
                  jax.ShapeDtypeStruct((n,), jnp.int32),
                  jax.ShapeDtypeStruct((n,), jnp.int32),
                  jax.ShapeDtypeStruct((n,), jnp.int32),
                  jax.ShapeDtypeStruct((n,), jnp.int32)),
        mesh=mesh,
        compiler_params=_SC_PARAMS_TILED,
        scratch_types=[
            pltpu.VMEM((2, D, 257), jnp.float32),
            pltpu.VMEM((2, D, 257), jnp.float32),
            pltpu.VMEM((2, 256 * D), jnp.float32),
            pltpu.VMEM((2, 256 * D), jnp.float32),
            pltpu.VMEM((tail * D // 128 if tail else 1, 128),
                       jnp.float32),
            pltpu.VMEM((2 * nch, CH), jnp.int32),
            pltpu.VMEM((per,), jnp.int32),
            pltpu.SemaphoreType.DMA((2,)),
            pltpu.SemaphoreType.DMA((2,)),
            pltpu.SemaphoreType.DMA,
        ],
    )
    def prep(ut_hbm, it_hbm, tu_hbm, ti_hbm, nu_hbm, ni_hbm,
             ulin, ilin, mub, mur, mib, mir,
             sbu, sbi, obu, obi, tailb, idx2, bvals,
             sem_src, sem_out, sem_sc):
        w = _widx()
        start = w * base_b + jnp.minimum(w, rem_b)

        nbase = w * per
        for c in range(nch):
            pltpu.sync_copy(nu_hbm.at[pl.ds(nbase + c * CH, CH)], idx2.at[c])
            pltpu.sync_copy(ni_hbm.at[pl.ds(nbase + c * CH, CH)],
                            idx2.at[nch + c])
        for g in range(per // L):
            bvals[pl.ds(g * L, L)] = nbase + g * L + _iota()
        map_copies = []
        for c in range(nch):
            map_copies.append(pltpu.async_copy(
                bvals.at[pl.ds(c * CH, CH)], mub.at[idx2.at[c]], sem_sc))
            map_copies.append(pltpu.async_copy(
                idx2.at[c], mur.at[idx2.at[c]], sem_sc))
            map_copies.append(pltpu.async_copy(
                bvals.at[pl.ds(c * CH, CH)], mib.at[idx2.at[nch + c]],
                sem_sc))
            map_copies.append(pltpu.async_copy(
                idx2.at[nch + c], mir.at[idx2.at[nch + c]], sem_sc))

        def src_fire(pb, blk):
            off = blk * BU
            pltpu.async_copy(ut_hbm.at[:, pl.ds(off, BU)],
                             sbu.at[pb, :, pl.ds(0, BU)], sem_src.at[pb])
            pltpu.async_copy(it_hbm.at[:, pl.ds(off, BU)],
                             sbi.at[pb, :, pl.ds(0, BU)], sem_src.at[pb])

        def src_wait(pb):
            pltpu.make_async_copy(ut_hbm.at[:, pl.ds(0, BU)],
                                  sbu.at[pb, :, pl.ds(0, BU)],
                                  sem_src.at[pb]).wait()
            pltpu.make_async_copy(it_hbm.at[:, pl.ds(0, BU)],
                                  sbi.at[pb, :, pl.ds(0, BU)],
                                  sem_src.at[pb]).wait()

        def out_fire(pb, blk):
            off = blk * (BU * D)
            pltpu.async_copy(obu.at[pb], ulin.at[pl.ds(off, BU * D)],
                             sem_out.at[pb])
            pltpu.async_copy(obi.at[pb], ilin.at[pl.ds(off, BU * D)],
                             sem_out.at[pb])

        def out_wait(pb):
            pltpu.make_async_copy(obu.at[pb], ulin.at[pl.ds(0, BU * D)],
                                  sem_out.at[pb]).wait()
            pltpu.make_async_copy(obi.at[pb], ilin.at[pl.ds(0, BU * D)],
                                  sem_out.at[pb]).wait()

        def transpose(pb):
            def ubody(k, _):
                f0 = _iota()
                f1 = f0 + D // 2
                for half in range(2):
                    u = k + k + half
                    uu = jnp.full((L,), 0, jnp.int32) + u
                    gu0 = plsc.load_gather(sbu.at[pb], [f0, uu])
                    gu1 = plsc.load_gather(sbu.at[pb], [f1, uu])
                    gi0 = plsc.load_gather(sbi.at[pb], [f0, uu])
                    gi1 = plsc.load_gather(sbi.at[pb], [f1, uu])
                    ob = u * D
                    obu[pb, pl.ds(ob, L)] = gu0
                    obu[pb, pl.ds(ob + L, L)] = gu1
                    obi[pb, pl.ds(ob, L)] = gi0
                    obi[pb, pl.ds(ob + L, L)] = gi1
                return 0

            lax.fori_loop(0, BU // 2, ubody, 0)

        src_fire(0, start)
        src_fire(1, start + 1)

        def step(t, _):
            wa = start + 2 * t
            pl.when(t > 0)(lambda: out_wait(0))
            pl.when(t > 0)(lambda: out_wait(1))
            src_wait(0)
            transpose(0)
            out_fire(0, wa)
            src_fire(0, jnp.minimum(wa + 2, start + base_b - 2))
            src_wait(1)
            transpose(1)
            out_fire(1, wa + 1)
            src_fire(1, jnp.minimum(wa + 3, start + base_b - 1))
            return 0

        lax.fori_loop(0, steps, step, 0)
        src_wait(0)
        src_wait(1)
        out_wait(0)
        out_wait(1)

        @pl.when(w < rem_b)
        def _():
            win = start + base_b
            src_fire(0, win)
            src_wait(0)
            transpose(0)
            out_fire(0, win)
            out_wait(0)

        @pl.when(w == NW - 1)
        def _():
            if tail:
                rows = tail * D // 128
                obase = full_b * BU * D
                pltpu.sync_copy(tu_hbm, tailb)
                for k in range(rows):
                    pltpu.sync_copy(tailb.at[k],
                                    ulin.at[pl.ds(obase + k * 128, 128)])
                pltpu.sync_copy(ti_hbm, tailb)
                for k in range(rows):
                    pltpu.sync_copy(tailb.at[k],
                                    ilin.at[pl.ds(obase + k * 128, 128)])

        for cp in map_copies:
            cp.wait()

    return prep(ut, itt, tail_u, tail_i, nei_users, nei_items)


def _score(user_mem, item_mem, p_u, p_i, mub, mur, mib, mir,
           pairs_u, pairs_i):
    P = pairs_u.shape[0]
    D = user_mem.shape[1]
    Bu = p_u.shape[0]
    Bi = p_i.shape[0]
    per = P // NW
    CH = 128
    nch = per // CH

    mesh = plsc.VectorSubcoreMesh(core_axis_name="c", subcore_axis_name="s")

    @functools.partial(
        pl.kernel,
        out_type=jax.ShapeDtypeStruct((P,), jnp.float32),
        mesh=mesh,
        compiler_params=_SC_PARAMS,
        scratch_types=[
            pltpu.VMEM((2, CH), jnp.int32),
            pltpu.VMEM((2, CH), jnp.int32),
            pltpu.VMEM((2, CH), jnp.int32),
            pltpu.VMEM((2, CH), jnp.int32),
            pltpu.VMEM((2, CH), jnp.int32),
            pltpu.VMEM((2, CH), jnp.int32),
            pltpu.VMEM((2, CH), jnp.int32),
            pltpu.VMEM((2, CH), jnp.int32),
            pltpu.VMEM((2, CH), jnp.float32),
            pltpu.VMEM((2, CH), jnp.float32),
            pltpu.VMEM((2, CH, 32), jnp.float32),
            pltpu.VMEM((2, CH, 32), jnp.float32),
            pltpu.VMEM((2, CH, 32), jnp.float32),
            pltpu.VMEM((2, CH, 32), jnp.float32),
            pltpu.VMEM((CH,), jnp.float32),
            pltpu.SemaphoreType.DMA((2,)),
            pltpu.SemaphoreType.DMA((2,)),
            pltpu.SemaphoreType.DMA((2,)),
        ],
    )
    def score(user_hbm, item_hbm, pu_hbm, pi_hbm,
              mub_hbm, mur_hbm, mib_hbm, mir_hbm,
              pru_hbm, pri_hbm, out_hbm,
              puv2, piv2, jub2, jur2, jib2, jir2, dbu2, dbi2, msku2, mski2,
              urows2, irows2, durows2, dirows2, scv,
              sem_map, sem_base, sem_delta):
        tbase = _widx() * per

        def front(pb, gb):
            pltpu.sync_copy(pru_hbm.at[pl.ds(gb, CH)], puv2.at[pb])
            pltpu.sync_copy(pri_hbm.at[pl.ds(gb, CH)], piv2.at[pb])
            pltpu.async_copy(mub_hbm.at[puv2.at[pb]], jub2.at[pb],
                             sem_map.at[pb])
            pltpu.async_copy(mur_hbm.at[puv2.at[pb]], jur2.at[pb],
                             sem_map.at[pb])
            pltpu.async_copy(mib_hbm.at[piv2.at[pb]], jib2.at[pb],
                             sem_map.at[pb])
            pltpu.async_copy(mir_hbm.at[piv2.at[pb]], jir2.at[pb],
                             sem_map.at[pb])
            pltpu.async_copy(user_hbm.at[puv2.at[pb]], urows2.at[pb],
                             sem_base.at[pb])
            pltpu.async_copy(item_hbm.at[piv2.at[pb]], irows2.at[pb],
                             sem_base.at[pb])

        def wait_map(pb):
            pltpu.make_async_copy(mub_hbm.at[puv2.at[pb]], jub2.at[pb],
                                  sem_map.at[pb]).wait()
            pltpu.make_async_copy(mur_hbm.at[puv2.at[pb]], jur2.at[pb],
                                  sem_map.at[pb]).wait()
            pltpu.make_async_copy(mib_hbm.at[piv2.at[pb]], jib2.at[pb],
                                  sem_map.at[pb]).wait()
            pltpu.make_async_copy(mir_hbm.at[piv2.at[pb]], jir2.at[pb],
                                  sem_map.at[pb]).wait()

        def wait_base(pb):
            pltpu.make_async_copy(user_hbm.at[puv2.at[pb]], urows2.at[pb],
                                  sem_base.at[pb]).wait()
            pltpu.make_async_copy(item_hbm.at[piv2.at[pb]], irows2.at[pb],
                                  sem_base.at[pb]).wait()

        def verify_and_fire_delta(pb):
            for g in range(CH // L):
                sl = pl.ds(g * L, L)
                hu = jur2[pb, sl] == puv2[pb, sl]
                buc = jnp.minimum(jnp.maximum(jub2[pb, sl], 0), Bu - 1)
                dbu2[pb, sl] = jnp.where(hu, buc, -1)
                msku2[pb, sl] = jnp.where(hu, STEP, 0.0)
                hi = jir2[pb, sl] == piv2[pb, sl]
                bic = jnp.minimum(jnp.maximum(jib2[pb, sl], 0), Bi - 1)
                dbi2[pb, sl] = jnp.where(hi, bic, -1)
                mski2[pb, sl] = jnp.where(hi, STEP, 0.0)
            pltpu.async_copy(
                pu_hbm.at[sc_core.Indices(dbu2.at[pb], ignored_value=-1)],
                durows2.at[pb], sem_delta.at[pb])
            pltpu.async_copy(
                pi_hbm.at[sc_core.Indices(dbi2.at[pb], ignored_value=-1)],
                dirows2.at[pb], sem_delta.at[pb])

        def wait_delta(pb):
            pltpu.make_async_copy(
                pu_hbm.at[sc_core.Indices(dbu2.at[pb], ignored_value=-1)],
                durows2.at[pb], sem_delta.at[pb]).wait()
            pltpu.make_async_copy(
                pi_hbm.at[sc_core.Indices(dbi2.at[pb], ignored_value=-1)],
                dirows2.at[pb], sem_delta.at[pb]).wait()

        def dots(pb, gb):
            def group_body(g, _):
                sl = pl.ds(g * L, L)
                rows = g * L + _iota()
                msku = msku2[pb, sl]
                mski = mski2[pb, sl]
                acc = jnp.zeros((L,), jnp.float32)
                for j in range(D):
                    cj = jnp.full((L,), j, jnp.int32)
                    cu = plsc.load_gather(urows2.at[pb], [rows, cj])
                    du = plsc.load_gather(durows2.at[pb], [rows, cj])
                    ci = plsc.load_gather(irows2.at[pb], [rows, cj])
                    di = plsc.load_gather(dirows2.at[pb], [rows, cj])
                    acc = acc + (cu + msku * du) * (ci + mski * di)
                scv[sl] = acc
                return 0

            lax.fori_loop(0, CH // L, group_body, 0)
            pltpu.sync_copy(scv, out_hbm.at[pl.ds(gb, CH)])

        front(0, tbase)

        def step(t, _):
            ga = tbase + (2 * t) * CH
            gb = ga + CH
            gnext = jnp.minimum(gb + CH, tbase + (nch - 1) * CH)
            wait_map(0)
            verify_and_fire_delta(0)
            front(1, gb)
            wait_base(0)
            wait_delta(0)
            dots(0, ga)
            wait_map(1)
            verify_and_fire_delta(1)
            front(0, gnext)
            wait_base(1)
            wait_delta(1)
            dots(1, gb)
            return 0

        lax.fori_loop(0, nch // 2, step, 0)
        wait_map(0)
        wait_base(0)

    return score(user_mem, item_mem, p_u, p_i, mub, mur, mib, mir,
                 pairs_u, pairs_i)


def kernel(user_mem, item_mem, p, nei_users, nei_items, pairs_u, pairs_i):
    n, d = user_mem.shape
    Bu = nei_users.shape[0]
    p_u = p[: Bu * d].reshape(Bu, d)
    p_i = p[Bu * d:].reshape(-1, d)
    full = (n // 128) * 128
    tail_u = user_mem[full:].reshape(-1, 128)
    tail_i = item_mem[full:].reshape(-1, 128)
    ulin, ilin, mub, mur, mib, mir = _prep(user_mem.T, item_mem.T,
                                           tail_u, tail_i,
                                           nei_users, nei_items)
    return _score(ulin.reshape(n, d), ilin.reshape(n, d), p_u, p_i,
                  mub, mur, mib, mir, pairs_u, pairs_i)


# speedup vs baseline: 1.4742x; 1.4742x over previous
import functools

import jax
import jax.numpy as jnp
from jax import lax
from jax.experimental import pallas as pl
from jax.experimental.pallas import tpu as pltpu
from jax.experimental.pallas import tpu_sc as plsc
from jax._src.pallas.mosaic import sc_core

NC = 2
NS = 16
NW = NC * NS
L = 16
STEP = 1.0 / 65536.0

_SC_PARAMS = pltpu.CompilerParams(
    use_tc_tiling_on_sc=False,
    needs_layout_passes=False,
)


def _widx():
    return lax.axis_index("s") * NC + lax.axis_index("c")


def _iota():
    return lax.iota(jnp.int32, L)


def _build_maps(nei_users, nei_items, n_users, n_items):
    Bn = nei_users.shape[0]
    per = Bn // NW
    CH = 128
    nch = per // CH

    mesh = plsc.VectorSubcoreMesh(core_axis_name="c", subcore_axis_name="s")

    @functools.partial(
        pl.kernel,
        out_type=(jax.ShapeDtypeStruct((n_users, 2), jnp.int32),
                  jax.ShapeDtypeStruct((n_items, 2), jnp.int32)),
        mesh=mesh,
        compiler_params=_SC_PARAMS,
        scratch_types=[
            pltpu.VMEM((2 * nch, CH), jnp.int32),
            pltpu.VMEM((per, 2), jnp.int32),
            pltpu.VMEM((per, 2), jnp.int32),
            pltpu.SemaphoreType.DMA,
        ],
    )
    def build(nei_u_hbm, nei_i_hbm, map_u_hbm, map_i_hbm,
              idx2, vals_u, vals_i, sem):
        base = _widx() * per
        for c in range(nch):
            pltpu.sync_copy(nei_u_hbm.at[pl.ds(base + c * CH, CH)], idx2.at[c])
            pltpu.sync_copy(nei_i_hbm.at[pl.ds(base + c * CH, CH)],
                            idx2.at[nch + c])
        z = jnp.zeros((L,), jnp.int32)
        for g in range(per // L):
            rows = g * L + _iota()
            bvec = base + g * L + _iota()
            c, off = (g * L) // CH, (g * L) % CH
            ru = idx2[c, pl.ds(off, L)]
            ri = idx2[nch + c, pl.ds(off, L)]
            plsc.store_scatter(vals_u, [rows, z], bvec)
            plsc.store_scatter(vals_u, [rows, z + 1], ru)
            plsc.store_scatter(vals_i, [rows, z], bvec)
            plsc.store_scatter(vals_i, [rows, z + 1], ri)
        copies = []
        for c in range(nch):
            copies.append(pltpu.async_copy(
                vals_u.at[pl.ds(c * CH, CH)], map_u_hbm.at[idx2.at[c]], sem))
            copies.append(pltpu.async_copy(
                vals_i.at[pl.ds(c * CH, CH)], map_i_hbm.at[idx2.at[nch + c]],
                sem))
        for cp in copies:
            cp.wait()

    return build(nei_users, nei_items)


def _score(user_mem, item_mem, p_u, p_i, map_u, map_i, pairs_u, pairs_i):
    P = pairs_u.shape[0]
    D = user_mem.shape[1]
    Bu = p_u.shape[0]
    Bi = p_i.shape[0]
    per = P // NW
    CH = 128
    nch = per // CH

    mesh = plsc.VectorSubcoreMesh(core_axis_name="c", subcore_axis_name="s")

    @functools.partial(
        pl.kernel,
        out_type=jax.ShapeDtypeStruct((P,), jnp.float32),
        mesh=mesh,
        compiler_params=_SC_PARAMS,
        scratch_types=[
            pltpu.VMEM((2, CH), jnp.int32),
            pltpu.VMEM((2, CH), jnp.int32),
            pltpu.VMEM((2, CH, 2), jnp.int32),
            pltpu.VMEM((2, CH, 2), jnp.int32),
            pltpu.VMEM((2, CH), jnp.int32),
            pltpu.VMEM((2, CH), jnp.int32),
            pltpu.VMEM((2, CH), jnp.float32),
            pltpu.VMEM((2, CH), jnp.float32),
            pltpu.VMEM((2, CH, 32), jnp.float32),
            pltpu.VMEM((2, CH, 32), jnp.float32),
            pltpu.VMEM((2, CH, 32), jnp.float32),
            pltpu.VMEM((2, CH, 32), jnp.float32),
            pltpu.VMEM((CH,), jnp.float32),
            pltpu.SemaphoreType.DMA((2,)),
            pltpu.SemaphoreType.DMA((2,)),
            pltpu.SemaphoreType.DMA((2,)),
        ],
    )
    def score(user_hbm, item_hbm, pu_hbm, pi_hbm, mu_hbm, mi_hbm,
              pru_hbm, pri_hbm, out_hbm,
              puv2, piv2, ju2, ji2, dbu2, dbi2, msku2, mski2,
              urows2, irows2, durows2, dirows2, scv,
              sem_map, sem_base, sem_delta):
        tbase = _widx() * per

        def front(pb, gb):
            pltpu.sync_copy(pru_hbm.at[pl.ds(gb, CH)], puv2.at[pb])
            pltpu.sync_copy(pri_hbm.at[pl.ds(gb, CH)], piv2.at[pb])
            pltpu.async_copy(mu_hbm.at[puv2.at[pb]], ju2.at[pb],
                             sem_map.at[pb])
            pltpu.async_copy(mi_hbm.at[piv2.at[pb]], ji2.at[pb],
                             sem_map.at[pb])
            pltpu.async_copy(user_hbm.at[puv2.at[pb]], urows2.at[pb],
                             sem_base.at[pb])
            pltpu.async_copy(item_hbm.at[piv2.at[pb]], irows2.at[pb],
                             sem_base.at[pb])

        def wait_map(pb):
            pltpu.make_async_copy(mu_hbm.at[puv2.at[pb]], ju2.at[pb],
                                  sem_map.at[pb]).wait()
            pltpu.make_async_copy(mi_hbm.at[piv2.at[pb]], ji2.at[pb],
                                  sem_map.at[pb]).wait()

        def wait_base(pb):
            pltpu.make_async_copy(user_hbm.at[puv2.at[pb]], urows2.at[pb],
                                  sem_base.at[pb]).wait()
            pltpu.make_async_copy(item_hbm.at[piv2.at[pb]], irows2.at[pb],
                                  sem_base.at[pb]).wait()

        def verify_and_fire_delta(pb):
            z = jnp.zeros((L,), jnp.int32)
            for g in range(CH // L):
                sl = pl.ds(g * L, L)
                rows = g * L + _iota()
                bu = plsc.load_gather(ju2.at[pb], [rows, z])
                ru = plsc.load_gather(ju2.at[pb], [rows, z + 1])
                hu = ru == puv2[pb, sl]
                buc = jnp.minimum(jnp.maximum(bu, 0), Bu - 1)
                dbu2[pb, sl] = jnp.where(hu, buc, -1)
                msku2[pb, sl] = jnp.where(hu, STEP, 0.0)
                bi = plsc.load_gather(ji2.at[pb], [rows, z])
                ri = plsc.load_gather(ji2.at[pb], [rows, z + 1])
                hi = ri == piv2[pb, sl]
                bic = jnp.minimum(jnp.maximum(bi, 0), Bi - 1)
                dbi2[pb, sl] = jnp.where(hi, bic, -1)
                mski2[pb, sl] = jnp.where(hi, STEP, 0.0)
            pltpu.async_copy(
                pu_hbm.at[sc_core.Indices(dbu2.at[pb], ignored_value=-1)],
                durows2.at[pb], sem_delta.at[pb])
            pltpu.async_copy(
                pi_hbm.at[sc_core.Indices(dbi2.at[pb], ignored_value=-1)],
                dirows2.at[pb], sem_delta.at[pb])

        def wait_delta(pb):
            pltpu.make_async_copy(
                pu_hbm.at[sc_core.Indices(dbu2.at[pb], ignored_value=-1)],
                durows2.at[pb], sem_delta.at[pb]).wait()
            pltpu.make_async_copy(
                pi_hbm.at[sc_core.Indices(dbi2.at[pb], ignored_value=-1)],
                dirows2.at[pb], sem_delta.at[pb]).wait()

        def dots(pb, gb):
            def group_body(g, _):
                sl = pl.ds(g * L, L)
                rows = g * L + _iota()
                msku = msku2[pb, sl]
                mski = mski2[pb, sl]
                acc = jnp.zeros((L,), jnp.float32)
                for j in range(D):
                    cj = jnp.full((L,), j, jnp.int32)
                    cu = plsc.load_gather(urows2.at[pb], [rows, cj])
                    du = plsc.load_gather(durows2.at[pb], [rows, cj])
                    ci = plsc.load_gather(irows2.at[pb], [rows, cj])
                    di = plsc.load_gather(dirows2.at[pb], [rows, cj])
                    acc = acc + (cu + msku * du) * (ci + mski * di)
                scv[sl] = acc
                return 0

            lax.fori_loop(0, CH // L, group_body, 0)
            pltpu.sync_copy(scv, out_hbm.at[pl.ds(gb, CH)])

        front(0, tbase)

        def step(t, _):
            ga = tbase + (2 * t) * CH
            gb = ga + CH
            gnext = jnp.minimum(gb + CH, tbase + (nch - 1) * CH)
            wait_map(0)
            verify_and_fire_delta(0)
            front(1, gb)
            wait_base(0)
            wait_delta(0)
            dots(0, ga)
            wait_map(1)
            verify_and_fire_delta(1)
            front(0, gnext)
            wait_base(1)
            wait_delta(1)
            dots(1, gb)
            return 0

        lax.fori_loop(0, nch // 2, step, 0)
        wait_map(0)
        wait_base(0)

    return score(user_mem, item_mem, p_u, p_i, map_u, map_i,
                 pairs_u, pairs_i)


def kernel(user_mem, item_mem, p, nei_users, nei_items, pairs_u, pairs_i):
    d = user_mem.shape[1]
    Bu = nei_users.shape[0]
    p_u = p[: Bu * d].reshape(Bu, d)
    p_i = p[Bu * d:].reshape(-1, d)
    map_u, map_i = _build_maps(nei_users, nei_items,
                               user_mem.shape[0], item_mem.shape[0])
    return _score(user_mem, item_mem, p_u, p_i, map_u, map_i,
                  pairs_u, pairs_i)
